# baseline (speedup 1.0000x reference)
import functools

import jax
import jax.numpy as jnp
from jax import lax
from jax.experimental import pallas as pl
from jax.experimental.pallas import tpu as pltpu
from jax.experimental.pallas import tpu_sc as plsc

B = 1024
CONF_DIM = 128
TREAT_DIM = 32
HID = 256
EMB = 32
TOPK = 16
OUT_DIM = 1
CORPUS = 100000
CHUNK = 2048
NCHUNK = (CORPUS + CHUNK - 1) // CHUNK
PADDED = NCHUNK * CHUNK
G = 128
NGROUP = PADDED // G
GPC = CHUNK // G
NCAND = TOPK * G
QB = 256
NQB = B // QB


def _encoder_body(patient_ref, treatment_ref, conf_ref,
                  wp1, bp1, wp2, bp2, wt, bt, wc, bc,
                  wce1, bce1, wce2, bce2,
                  t_emb_ref, c_emb_ref, ce_ref):
    f32 = jnp.float32
    patient = patient_ref[...]
    pe1 = jnp.maximum(jnp.dot(patient, wp1[...], preferred_element_type=f32) + bp1[...], 0.0)
    patient_emb = jnp.dot(pe1, wp2[...], preferred_element_type=f32) + bp2[...]
    t_emb = jnp.dot(treatment_ref[...], wt[...], preferred_element_type=f32) + bt[...]
    c_emb = jnp.dot(conf_ref[...], wc[...], preferred_element_type=f32) + bc[...]
    h = jnp.concatenate([patient_emb, c_emb], axis=1)
    h1 = jnp.maximum(jnp.dot(h, wce1[...], preferred_element_type=f32) + bce1[...], 0.0)
    ce = jnp.dot(h1, wce2[...], preferred_element_type=f32) + bce2[...]
    t_emb_ref[...] = t_emb
    c_emb_ref[...] = c_emb
    ce_ref[...] = ce


def _sim_gm_body(qn_ref, qnt_ref, corpus_ref, sims_ref, gm_ref):
    i = pl.program_id(0)
    f32 = jnp.float32
    cn = corpus_ref[...]
    sims_ref[...] = jnp.dot(qn_ref[...], cn.T, preferred_element_type=f32)
    st = jnp.dot(cn, qnt_ref[...], preferred_element_type=f32)

    @pl.when(i < NCHUNK - 1)
    def _():
        gm_ref[...] = jnp.max(st.reshape(GPC, G, B), axis=1)

    @pl.when(i == NCHUNK - 1)
    def _():
        gi = lax.broadcasted_iota(jnp.int32, (CHUNK, 1), 0) + (NCHUNK - 1) * CHUNK
        stm = jnp.where(gi < CORPUS, st, -jnp.inf)
        gm_ref[...] = jnp.max(stm.reshape(GPC, G, B), axis=1)


def _group_topk_body(gm_ref, gid_out_ref):
    gm = gm_ref[...]
    gi = lax.broadcasted_iota(jnp.int32, (NGROUP, B), 0)
    a = gm
    sidxs = []
    for _ in range(TOPK):
        m = jnp.max(a, axis=0, keepdims=True)
        sel = a == m
        am = jnp.min(jnp.where(sel, gi, jnp.int32(2**30)), axis=0, keepdims=True)
        sidxs.append(am)
        a = jnp.where(sel & (gi == am), -jnp.inf, a)
    gid_out_ref[...] = jnp.concatenate(sidxs, axis=0)


def _final_topk_body(vals_ref, gidx_ref, s_out_ref, i_out_ref):
    gidx = gidx_ref[...]
    a = jnp.where(gidx < CORPUS, vals_ref[...], -jnp.inf)
    svals, sidxs = [], []
    for _ in range(TOPK):
        m = jnp.max(a, axis=1, keepdims=True)
        sel = a == m
        am = jnp.min(jnp.where(sel, gidx, jnp.int32(2**30)), axis=1, keepdims=True)
        svals.append(m)
        sidxs.append(am)
        a = jnp.where(sel & (gidx == am), -jnp.inf, a)
    pad = ((0, 0), (0, 128 - TOPK))
    s_out_ref[...] = jnp.pad(jnp.concatenate(svals, axis=1), pad)
    i_out_ref[...] = jnp.pad(jnp.concatenate(sidxs, axis=1), pad)


def _head_body(retr_ref, t_emb_ref, c_emb_ref,
               wr, br, wo1, bo1, wo2, bo2, wo3, bo3, out_ref):
    f32 = jnp.float32
    enc = jnp.dot(retr_ref[...], wr[...], preferred_element_type=f32) + br[...]
    h1 = (jnp.dot(t_emb_ref[...], wo1[0:HID, :], preferred_element_type=f32)
          + jnp.dot(c_emb_ref[...], wo1[HID:2 * HID, :], preferred_element_type=f32)
          + jnp.dot(enc, wo1[2 * HID:3 * HID, :], preferred_element_type=f32)
          + bo1[...])
    h1 = jnp.maximum(h1, 0.0)
    h2 = jnp.maximum(jnp.dot(h1, wo2[...], preferred_element_type=f32) + bo2[...], 0.0)
    out_ref[...] = jnp.dot(h2, wo3[...], preferred_element_type=f32) + bo3[...]


def _sc_gather_rows(table, flat_idx):
    info = plsc.get_sparse_core_info()
    nc, ns = info.num_cores, info.num_subcores
    nw = nc * ns
    n = flat_idx.shape[0]
    bpw = n // nw
    d = table.shape[1]
    if d % 128 != 0:
        d = 128
        table = jnp.pad(table, ((0, 0), (0, d - table.shape[1])))
    mesh = plsc.VectorSubcoreMesh(core_axis_name="c", subcore_axis_name="s")

    @functools.partial(
        pl.kernel, mesh=mesh,
        out_type=jax.ShapeDtypeStruct((n, d), jnp.float32),
        scratch_types=[
            pltpu.VMEM((bpw,), jnp.int32),
            pltpu.VMEM((bpw, d), jnp.float32),
            pltpu.SemaphoreType.DMA,
        ],
    )
    def k(table_hbm, idx_hbm, out_hbm, idx_v, rows_v, sem):
        wid = lax.axis_index("s") * nc + lax.axis_index("c")
        base = wid * bpw
        pltpu.sync_copy(idx_hbm.at[pl.ds(base, bpw)], idx_v)
        pltpu.async_copy(table_hbm.at[idx_v], rows_v, sem).wait()
        pltpu.sync_copy(rows_v, out_hbm.at[pl.ds(base, bpw)])

    return k(table, flat_idx)


def kernel(patient, treatment, confounders, corpus_embeddings,
           W_p1, b_p1, W_p2, b_p2, W_t, b_t, W_c, b_c,
           W_ce1, b_ce1, W_ce2, b_ce2, W_r, b_r,
           W_o1, b_o1, W_o2, b_o2, W_o3, b_o3):
    f32 = jnp.float32
    i32 = jnp.int32
    r2 = lambda b: b.reshape(1, -1)

    t_emb, c_emb, ce = pl.pallas_call(
        _encoder_body,
        out_shape=[
            jax.ShapeDtypeStruct((B, HID), f32),
            jax.ShapeDtypeStruct((B, HID), f32),
            jax.ShapeDtypeStruct((B, EMB), f32),
        ],
    )(patient, treatment, confounders,
      W_p1, r2(b_p1), W_p2, r2(b_p2), W_t, r2(b_t), W_c, r2(b_c),
      W_ce1, r2(b_ce1), W_ce2, r2(b_ce2))

    qn = ce / jnp.clip(jnp.linalg.norm(ce, axis=1, keepdims=True), 1e-12)
    qnt = qn.T
    corpus_n = corpus_embeddings / jnp.clip(
        jnp.linalg.norm(corpus_embeddings, axis=1, keepdims=True), 1e-12)
    corpus_pad = jnp.pad(corpus_n, ((0, PADDED - CORPUS), (0, 0)))
    sims, gm = pl.pallas_call(
        _sim_gm_body,
        grid=(NCHUNK,),
        in_specs=[
            pl.BlockSpec((B, EMB), lambda i: (0, 0)),
            pl.BlockSpec((EMB, B), lambda i: (0, 0)),
            pl.BlockSpec((CHUNK, EMB), lambda i: (i, 0)),
        ],
        out_specs=[
            pl.BlockSpec((B, CHUNK), lambda i: (0, i)),
            pl.BlockSpec((GPC, B), lambda i: (i, 0)),
        ],
        out_shape=[
            jax.ShapeDtypeStruct((B, PADDED), f32),
            jax.ShapeDtypeStruct((NGROUP, B), f32),
        ],
        compiler_params=pltpu.CompilerParams(dimension_semantics=("arbitrary",)),
    )(qn, qnt, corpus_pad)

    gid_t = pl.pallas_call(
        _group_topk_body,
        out_shape=jax.ShapeDtypeStruct((TOPK, B), i32),
    )(gm)

    gid = gid_t.T
    slab_idx = (jnp.arange(B, dtype=i32)[:, None] * NGROUP + gid).reshape(-1)
    cand = _sc_gather_rows(sims.reshape(B * NGROUP, G), slab_idx)
    gidx_full = (gid[:, :, None] * G
                 + jnp.arange(G, dtype=i32)[None, None, :]).reshape(B, NCAND)

    scores_pad, idx_pad = pl.pallas_call(
        _final_topk_body,
        grid=(NQB,),
        in_specs=[
            pl.BlockSpec((QB, NCAND), lambda i: (i, 0)),
            pl.BlockSpec((QB, NCAND), lambda i: (i, 0)),
        ],
        out_specs=[
            pl.BlockSpec((QB, 128), lambda i: (i, 0)),
            pl.BlockSpec((QB, 128), lambda i: (i, 0)),
        ],
        out_shape=[
            jax.ShapeDtypeStruct((B, 128), f32),
            jax.ShapeDtypeStruct((B, 128), i32),
        ],
        compiler_params=pltpu.CompilerParams(dimension_semantics=("arbitrary",)),
    )(cand.reshape(B, NCAND), gidx_full)
    scores = scores_pad[:, :TOPK]
    idx = idx_pad[:, :TOPK]

    retrieved = _sc_gather_rows(corpus_embeddings, idx.reshape(-1))[:, :EMB]
    outcome = pl.pallas_call(
        _head_body,
        out_shape=jax.ShapeDtypeStruct((B, OUT_DIM), f32),
    )(retrieved.reshape(B, TOPK * EMB), t_emb, c_emb,
      W_r, r2(b_r), W_o1, r2(b_o1), W_o2, r2(b_o2), W_o3, r2(b_o3))

    return outcome, scores, idx, ce


# speedup vs baseline: 1.6468x; 1.6468x over previous
import functools

import jax
import jax.numpy as jnp
from jax import lax
from jax.experimental import pallas as pl
from jax.experimental.pallas import tpu as pltpu
from jax.experimental.pallas import tpu_sc as plsc

B = 1024
CONF_DIM = 128
TREAT_DIM = 32
HID = 256
EMB = 32
TOPK = 16
OUT_DIM = 1
CORPUS = 100000
CHUNK = 2048
NCHUNK = (CORPUS + CHUNK - 1) // CHUNK
PADDED = NCHUNK * CHUNK
G = 128
NGROUP = PADDED // G
GPC = CHUNK // G
NCAND = TOPK * G
QB = 256
NQB = B // QB


def _encoder_body(patient_ref, treatment_ref, conf_ref,
                  wp1, bp1, wp2, bp2, wt, bt, wc, bc,
                  wce1, bce1, wce2, bce2,
                  t_emb_ref, c_emb_ref, ce_ref):
    f32 = jnp.float32
    patient = patient_ref[...]
    pe1 = jnp.maximum(jnp.dot(patient, wp1[...], preferred_element_type=f32) + bp1[...], 0.0)
    patient_emb = jnp.dot(pe1, wp2[...], preferred_element_type=f32) + bp2[...]
    t_emb = jnp.dot(treatment_ref[...], wt[...], preferred_element_type=f32) + bt[...]
    c_emb = jnp.dot(conf_ref[...], wc[...], preferred_element_type=f32) + bc[...]
    h = jnp.concatenate([patient_emb, c_emb], axis=1)
    h1 = jnp.maximum(jnp.dot(h, wce1[...], preferred_element_type=f32) + bce1[...], 0.0)
    ce = jnp.dot(h1, wce2[...], preferred_element_type=f32) + bce2[...]
    t_emb_ref[...] = t_emb
    c_emb_ref[...] = c_emb
    ce_ref[...] = ce


def _sim_gm_body(qn_ref, qnt_ref, corpus_ref, sims_ref, gm_ref):
    i = pl.program_id(0)
    f32 = jnp.float32
    cn = corpus_ref[...]
    s = jnp.dot(qn_ref[...], cn.T, preferred_element_type=f32)
    sims_ref[...] = s.reshape(B, GPC, G)
    st = jnp.dot(cn, qnt_ref[...], preferred_element_type=f32)
    gi = lax.broadcasted_iota(jnp.int32, (CHUNK, 1), 0) + i * CHUNK
    st = jnp.where(gi < CORPUS, st, -jnp.inf)
    gm_ref[...] = jnp.max(st.reshape(GPC, G, B), axis=1)


def _group_topk_body(gm_ref, gid_out_ref):
    gm = gm_ref[...]
    gi = lax.broadcasted_iota(jnp.int32, (NGROUP, B), 0)
    a = gm
    sidxs = []
    for _ in range(TOPK):
        m = jnp.max(a, axis=0, keepdims=True)
        sel = a == m
        am = jnp.min(jnp.where(sel, gi, jnp.int32(2**30)), axis=0, keepdims=True)
        sidxs.append(am)
        a = jnp.where(sel & (gi == am), -jnp.inf, a)
    gid_out_ref[...] = jnp.concatenate(sidxs, axis=0)


def _final_topk_body(vals_ref, gidx_ref, s_out_ref, i_out_ref):
    gidx = gidx_ref[...]
    a = jnp.where(gidx < CORPUS, vals_ref[...], -jnp.inf)
    svals, sidxs = [], []
    for _ in range(TOPK):
        m = jnp.max(a, axis=1, keepdims=True)
        sel = a == m
        am = jnp.min(jnp.where(sel, gidx, jnp.int32(2**30)), axis=1, keepdims=True)
        svals.append(m)
        sidxs.append(am)
        a = jnp.where(sel & (gidx == am), -jnp.inf, a)
    pad = ((0, 0), (0, 128 - TOPK))
    s_out_ref[...] = jnp.pad(jnp.concatenate(svals, axis=1), pad)
    i_out_ref[...] = jnp.pad(jnp.concatenate(sidxs, axis=1), pad)


def _head_body(retr_ref, t_emb_ref, c_emb_ref,
               wr, br, wo1, bo1, wo2, bo2, wo3, bo3, out_ref):
    f32 = jnp.float32
    enc = jnp.dot(retr_ref[...], wr[...], preferred_element_type=f32) + br[...]
    h1 = (jnp.dot(t_emb_ref[...], wo1[0:HID, :], preferred_element_type=f32)
          + jnp.dot(c_emb_ref[...], wo1[HID:2 * HID, :], preferred_element_type=f32)
          + jnp.dot(enc, wo1[2 * HID:3 * HID, :], preferred_element_type=f32)
          + bo1[...])
    h1 = jnp.maximum(h1, 0.0)
    h2 = jnp.maximum(jnp.dot(h1, wo2[...], preferred_element_type=f32) + bo2[...], 0.0)
    out_ref[...] = jnp.dot(h2, wo3[...], preferred_element_type=f32) + bo3[...]


def _sc_gather_rows(table, flat_idx):
    info = plsc.get_sparse_core_info()
    nc, ns = info.num_cores, info.num_subcores
    nw = nc * ns
    n = flat_idx.shape[0]
    bpw = n // nw
    d = table.shape[1]
    if d % 128 != 0:
        d = 128
        table = jnp.pad(table, ((0, 0), (0, d - table.shape[1])))
    mesh = plsc.VectorSubcoreMesh(core_axis_name="c", subcore_axis_name="s")

    @functools.partial(
        pl.kernel, mesh=mesh,
        out_type=jax.ShapeDtypeStruct((n, d), jnp.float32),
        scratch_types=[
            pltpu.VMEM((bpw,), jnp.int32),
            pltpu.VMEM((bpw, d), jnp.float32),
            pltpu.SemaphoreType.DMA,
        ],
    )
    def k(table_hbm, idx_hbm, out_hbm, idx_v, rows_v, sem):
        wid = lax.axis_index("s") * nc + lax.axis_index("c")
        base = wid * bpw
        pltpu.sync_copy(idx_hbm.at[pl.ds(base, bpw)], idx_v)
        pltpu.async_copy(table_hbm.at[idx_v], rows_v, sem).wait()
        pltpu.sync_copy(rows_v, out_hbm.at[pl.ds(base, bpw)])

    return k(table, flat_idx)


def kernel(patient, treatment, confounders, corpus_embeddings,
           W_p1, b_p1, W_p2, b_p2, W_t, b_t, W_c, b_c,
           W_ce1, b_ce1, W_ce2, b_ce2, W_r, b_r,
           W_o1, b_o1, W_o2, b_o2, W_o3, b_o3):
    f32 = jnp.float32
    i32 = jnp.int32
    r2 = lambda b: b.reshape(1, -1)

    t_emb, c_emb, ce = pl.pallas_call(
        _encoder_body,
        out_shape=[
            jax.ShapeDtypeStruct((B, HID), f32),
            jax.ShapeDtypeStruct((B, HID), f32),
            jax.ShapeDtypeStruct((B, EMB), f32),
        ],
    )(patient, treatment, confounders,
      W_p1, r2(b_p1), W_p2, r2(b_p2), W_t, r2(b_t), W_c, r2(b_c),
      W_ce1, r2(b_ce1), W_ce2, r2(b_ce2))

    qn = ce / jnp.clip(jnp.linalg.norm(ce, axis=1, keepdims=True), 1e-12)
    qnt = qn.T
    corpus_n = corpus_embeddings / jnp.clip(
        jnp.linalg.norm(corpus_embeddings, axis=1, keepdims=True), 1e-12)
    corpus_pad = jnp.pad(corpus_n, ((0, PADDED - CORPUS), (0, 0)))
    sims, gm = pl.pallas_call(
        _sim_gm_body,
        grid=(NCHUNK,),
        in_specs=[
            pl.BlockSpec((B, EMB), lambda i: (0, 0)),
            pl.BlockSpec((EMB, B), lambda i: (0, 0)),
            pl.BlockSpec((CHUNK, EMB), lambda i: (i, 0)),
        ],
        out_specs=[
            pl.BlockSpec((B, GPC, G), lambda i: (0, i, 0)),
            pl.BlockSpec((GPC, B), lambda i: (i, 0)),
        ],
        out_shape=[
            jax.ShapeDtypeStruct((B, NGROUP, G), f32),
            jax.ShapeDtypeStruct((NGROUP, B), f32),
        ],
        compiler_params=pltpu.CompilerParams(dimension_semantics=("arbitrary",)),
    )(qn, qnt, corpus_pad)

    gid_t = pl.pallas_call(
        _group_topk_body,
        out_shape=jax.ShapeDtypeStruct((TOPK, B), i32),
    )(gm)

    gid = gid_t.T
    slab_idx = (jnp.arange(B, dtype=i32)[:, None] * NGROUP + gid).reshape(-1)
    cand = _sc_gather_rows(sims.reshape(B * NGROUP, G), slab_idx)
    gidx_full = (gid[:, :, None] * G
                 + jnp.arange(G, dtype=i32)[None, None, :]).reshape(B, NCAND)

    scores_pad, idx_pad = pl.pallas_call(
        _final_topk_body,
        grid=(NQB,),
        in_specs=[
            pl.BlockSpec((QB, NCAND), lambda i: (i, 0)),
            pl.BlockSpec((QB, NCAND), lambda i: (i, 0)),
        ],
        out_specs=[
            pl.BlockSpec((QB, 128), lambda i: (i, 0)),
            pl.BlockSpec((QB, 128), lambda i: (i, 0)),
        ],
        out_shape=[
            jax.ShapeDtypeStruct((B, 128), f32),
            jax.ShapeDtypeStruct((B, 128), i32),
        ],
        compiler_params=pltpu.CompilerParams(dimension_semantics=("arbitrary",)),
    )(cand.reshape(B, NCAND), gidx_full)
    scores = scores_pad[:, :TOPK]
    idx = idx_pad[:, :TOPK]

    retrieved = _sc_gather_rows(corpus_embeddings, idx.reshape(-1))[:, :EMB]
    outcome = pl.pallas_call(
        _head_body,
        out_shape=jax.ShapeDtypeStruct((B, OUT_DIM), f32),
    )(retrieved.reshape(B, TOPK * EMB), t_emb, c_emb,
      W_r, r2(b_r), W_o1, r2(b_o1), W_o2, r2(b_o2), W_o3, r2(b_o3))

    return outcome, scores, idx, ce
